# baseline (speedup 1.0000x reference)
import functools

import jax
import jax.numpy as jnp
from jax import lax
from jax.experimental import pallas as pl
from jax.experimental.pallas import tpu as pltpu
from jax.experimental.pallas import tpu_sc as plsc

VOCAB = 100000
DIM = 128
B = 1024
L = 200
HID = 256
OUT = 5

LANES = 16
NC = 2
NS = 16
NW = NC * NS
B_PER_W = B // NW
DGRP = DIM // LANES
NBUF = 8


def _sc_pool(xt_hbm, emb_hbm, out_hbm, idx_flat, osum, *bufs_and_sems):
    rows_bufs = bufs_and_sems[:NBUF]
    sems = bufs_and_sems[NBUF:2 * NBUF]
    idx_sem = bufs_and_sems[2 * NBUF]
    wid = lax.axis_index("s") * NC + lax.axis_index("c")
    base = wid * B_PER_W

    def idx_start(l, carry):
        pltpu.async_copy(xt_hbm.at[l, pl.ds(base, B_PER_W)],
                         idx_flat.at[pl.ds(l * B_PER_W, B_PER_W)], idx_sem)
        return carry

    lax.fori_loop(0, L, idx_start, 0)

    def zero_row(j, carry):
        for d in range(DGRP):
            osum[j, pl.ds(d * LANES, LANES)] = jnp.zeros((LANES,), jnp.float32)
        return carry

    lax.fori_loop(0, B_PER_W, zero_row, 0)

    def idx_wait(l, carry):
        pltpu.make_async_copy(
            xt_hbm.at[l, pl.ds(base, B_PER_W)],
            idx_flat.at[pl.ds(l * B_PER_W, B_PER_W)], idx_sem).wait()
        return carry

    lax.fori_loop(0, L, idx_wait, 0)

    def start_gather(l, rows, sem):
        pltpu.async_copy(
            emb_hbm.at[idx_flat.at[pl.ds(l * B_PER_W, B_PER_W)]], rows, sem)

    def wait_gather(l, rows, sem):
        pltpu.make_async_copy(
            emb_hbm.at[idx_flat.at[pl.ds(l * B_PER_W, B_PER_W)]], rows, sem
        ).wait()

    def accumulate(rows):
        def acc_body(j, carry):
            for d in range(DGRP):
                plsc.addupdate(osum.at[j, pl.ds(d * LANES, LANES)],
                               rows[j, pl.ds(d * LANES, LANES)])
            return carry

        lax.fori_loop(0, B_PER_W, acc_body, 0)

    for b in range(NBUF - 1):
        start_gather(b, rows_bufs[b], sems[b])

    def body(g, carry):
        l0 = g * NBUF
        for b in range(NBUF):
            l = l0 + b

            @pl.when(l + NBUF - 1 < L)
            def _(b=b, l=l):
                nb = (b + NBUF - 1) % NBUF
                start_gather(l + NBUF - 1, rows_bufs[nb], sems[nb])

            wait_gather(l, rows_bufs[b], sems[b])
            accumulate(rows_bufs[b])
        return carry

    lax.fori_loop(0, L // NBUF, body, 0)
    pltpu.sync_copy(osum, out_hbm.at[pl.ds(base, B_PER_W)])


@jax.jit
def _pooled_sum(x, emb):
    mesh = plsc.VectorSubcoreMesh(core_axis_name="c", subcore_axis_name="s")
    f = functools.partial(
        pl.kernel,
        mesh=mesh,
        out_type=jax.ShapeDtypeStruct((B, DIM), jnp.float32),
        scratch_types=(
            [pltpu.VMEM((L * B_PER_W,), jnp.int32),
             pltpu.VMEM((B_PER_W, DIM), jnp.float32)]
            + [pltpu.VMEM((B_PER_W, DIM), jnp.float32) for _ in range(NBUF)]
            + [pltpu.SemaphoreType.DMA for _ in range(NBUF)]
            + [pltpu.SemaphoreType.DMA]
        ),
    )(_sc_pool)
    return f(jnp.transpose(x), emb)


def _mlp_body(pooled_ref, len_ref, w1_ref, b1_ref, w2_ref, b2_ref, out_ref):
    inv = 1.0 / len_ref[...].astype(jnp.float32)
    rep = pooled_ref[...] * inv
    h = jnp.dot(rep, w1_ref[...], preferred_element_type=jnp.float32)
    h = jnp.maximum(h + b1_ref[...], 0.0)
    out = jnp.dot(h, w2_ref[...], preferred_element_type=jnp.float32)
    out_ref[...] = out + b2_ref[...]


@jax.jit
def _mlp(pooled, lengths, W1, b1, W2, b2):
    return pl.pallas_call(
        _mlp_body,
        out_shape=jax.ShapeDtypeStruct((B, OUT), jnp.float32),
    )(pooled, lengths.reshape(B, 1), W1, b1.reshape(1, HID),
      W2, b2.reshape(1, OUT))


def kernel(x, lengths, emb, W1, b1, W2, b2):
    pooled = _pooled_sum(x, emb)
    return _mlp(pooled, lengths, W1, b1, W2, b2)


# speedup vs baseline: 1.3340x; 1.3340x over previous
import functools

import jax
import jax.numpy as jnp
from jax import lax
from jax.experimental import pallas as pl
from jax.experimental.pallas import tpu as pltpu
from jax.experimental.pallas import tpu_sc as plsc

VOCAB = 100000
DIM = 128
B = 1024
L = 200
HID = 256
OUT = 5

LANES = 16
NC = 2
NS = 16
NW = NC * NS
B_PER_W = B // NW
DGRP = DIM // LANES
CHUNKS = ((0, 64), (64, 64), (128, 72))


NBUF = 4


def _sc_pool(x_hbm, emb_hbm, out_hbm, idx_all, osum, *bufs_and_sems):
    rows_bufs = bufs_and_sems[:NBUF]
    sems = bufs_and_sems[NBUF:]
    wid = lax.axis_index("s") * NC + lax.axis_index("c")
    base = wid * B_PER_W

    pltpu.sync_copy(x_hbm.at[pl.ds(base, B_PER_W)], idx_all)

    def start_gather(i, rows, sem):
        for off, n in CHUNKS:
            pltpu.async_copy(
                emb_hbm.at[idx_all.at[i, pl.ds(off, n)]], rows.at[pl.ds(off, n)], sem)

    def wait_gather(i, rows, sem):
        for off, n in CHUNKS:
            pltpu.make_async_copy(
                emb_hbm.at[idx_all.at[i, pl.ds(off, n)]], rows.at[pl.ds(off, n)], sem
            ).wait()

    def reduce_row(i, rows):
        accs = tuple(jnp.zeros((LANES,), jnp.float32) for _ in range(DGRP))

        def red_body(l, accs):
            r0 = l * 8
            new = list(accs)
            for r in range(8):
                for d in range(DGRP):
                    new[d] = new[d] + rows[r0 + r, pl.ds(d * LANES, LANES)]
            return tuple(new)

        accs = lax.fori_loop(0, L // 8, red_body, accs)
        for d in range(DGRP):
            osum[i, pl.ds(d * LANES, LANES)] = accs[d]

    for b in range(NBUF - 1):
        start_gather(b, rows_bufs[b], sems[b])

    def body(g, carry):
        i0 = g * NBUF
        for b in range(NBUF):
            i = i0 + b

            @pl.when(i + NBUF - 1 < B_PER_W)
            def _(b=b, i=i):
                start_gather(i + NBUF - 1,
                             rows_bufs[(b + NBUF - 1) % NBUF],
                             sems[(b + NBUF - 1) % NBUF])

            wait_gather(i, rows_bufs[b], sems[b])
            reduce_row(i, rows_bufs[b])
        return carry

    lax.fori_loop(0, B_PER_W // NBUF, body, 0)
    pltpu.sync_copy(osum, out_hbm.at[pl.ds(base, B_PER_W)])


@jax.jit
def _pooled_sum(x, emb):
    mesh = plsc.VectorSubcoreMesh(core_axis_name="c", subcore_axis_name="s")
    f = functools.partial(
        pl.kernel,
        mesh=mesh,
        out_type=jax.ShapeDtypeStruct((B, DIM), jnp.float32),
        scratch_types=(
            [pltpu.VMEM((B_PER_W, L), jnp.int32),
             pltpu.VMEM((B_PER_W, DIM), jnp.float32)]
            + [pltpu.VMEM((L, DIM), jnp.float32) for _ in range(NBUF)]
            + [pltpu.SemaphoreType.DMA for _ in range(NBUF)]
        ),
    )(_sc_pool)
    return f(x, emb)


def _mlp_body(pooled_ref, len_ref, w1_ref, b1_ref, w2_ref, b2_ref, out_ref):
    inv = 1.0 / len_ref[...].astype(jnp.float32)
    rep = pooled_ref[...] * inv
    h = jnp.dot(rep, w1_ref[...], preferred_element_type=jnp.float32)
    h = jnp.maximum(h + b1_ref[...], 0.0)
    out = jnp.dot(h, w2_ref[...], preferred_element_type=jnp.float32)
    out_ref[...] = out + b2_ref[...]


@jax.jit
def _mlp(pooled, lengths, W1, b1, W2, b2):
    return pl.pallas_call(
        _mlp_body,
        out_shape=jax.ShapeDtypeStruct((B, OUT), jnp.float32),
    )(pooled, lengths.reshape(B, 1), W1, b1.reshape(1, HID),
      W2, b2.reshape(1, OUT))


def kernel(x, lengths, emb, W1, b1, W2, b2):
    pooled = _pooled_sum(x, emb)
    return _mlp(pooled, lengths, W1, b1, W2, b2)
